# speedup vs baseline: 1.5331x; 1.5331x over previous
import functools

import jax
import jax.numpy as jnp
from jax import lax
from jax.experimental import pallas as pl
from jax.experimental.pallas import tpu as pltpu
from jax.experimental.pallas import tpu_sc as plsc

EMB = 512
BATCH = 32
SEQ = 1024
NC = 2
NS = 16
NW = NC * NS
B = BATCH * SEQ
B_PER_W = B // NW
CHUNK = 128
N_CHUNK = B_PER_W // CHUNK

_mesh = plsc.VectorSubcoreMesh(core_axis_name="c", subcore_axis_name="s")


@functools.partial(
    pl.kernel,
    mesh=_mesh,
    out_type=jax.ShapeDtypeStruct((B, EMB), jnp.float32),
    scratch_types=[
        pltpu.VMEM((N_CHUNK, CHUNK), jnp.int32),
        pltpu.VMEM((CHUNK, EMB), jnp.float32),
        pltpu.SemaphoreType.DMA,
    ],
)
def _emb_lookup(idx_hbm, table_hbm, out_hbm, idx_v, rows_v, sem):
    wid = lax.axis_index("s") * NC + lax.axis_index("c")
    base = wid * B_PER_W
    pltpu.sync_copy(idx_hbm.at[wid], idx_v)
    for i in range(N_CHUNK):
        pltpu.async_copy(table_hbm.at[idx_v.at[i]], rows_v, sem).wait()
        pltpu.sync_copy(rows_v, out_hbm.at[pl.ds(base + i * CHUNK, CHUNK)])


def kernel(x, table):
    idx = x.astype(jnp.int32).reshape(NW, N_CHUNK, CHUNK)
    out = _emb_lookup(idx, table)
    return out.reshape(BATCH, SEQ, EMB)


# speedup vs baseline: 1.5575x; 1.0159x over previous
import functools

import jax
import jax.numpy as jnp
from jax import lax
from jax.experimental import pallas as pl
from jax.experimental.pallas import tpu as pltpu
from jax.experimental.pallas import tpu_sc as plsc

EMB = 512
BATCH = 32
SEQ = 1024
NC = 2
NS = 16
NW = NC * NS
B = BATCH * SEQ
B_PER_W = B // NW
CHUNK = 64
N_CHUNK = B_PER_W // CHUNK
NBUF = 2

_mesh = plsc.VectorSubcoreMesh(core_axis_name="c", subcore_axis_name="s")


@functools.partial(
    pl.kernel,
    mesh=_mesh,
    out_type=jax.ShapeDtypeStruct((B, EMB), jnp.float32),
    scratch_types=[
        pltpu.VMEM((N_CHUNK, CHUNK), jnp.int32),
        pltpu.VMEM((NBUF, CHUNK, EMB), jnp.float32),
        pltpu.SemaphoreType.DMA,
    ],
)
def _emb_lookup(idx_hbm, table_hbm, out_hbm, idx_v, rows_v, sem):
    wid = lax.axis_index("s") * NC + lax.axis_index("c")
    base = wid * B_PER_W
    pltpu.sync_copy(idx_hbm.at[wid], idx_v)
    gathers = [None] * N_CHUNK
    gathers[0] = pltpu.async_copy(table_hbm.at[idx_v.at[0]], rows_v.at[0], sem)
    for i in range(N_CHUNK):
        gathers[i].wait()
        if i + 1 < N_CHUNK:
            gathers[i + 1] = pltpu.async_copy(
                table_hbm.at[idx_v.at[i + 1]], rows_v.at[(i + 1) % NBUF], sem)
        pltpu.sync_copy(rows_v.at[i % NBUF],
                        out_hbm.at[pl.ds(base + i * CHUNK, CHUNK)])


def kernel(x, table):
    idx = x.astype(jnp.int32).reshape(NW, N_CHUNK, CHUNK)
    out = _emb_lookup(idx, table)
    return out.reshape(BATCH, SEQ, EMB)


# speedup vs baseline: 1.5952x; 1.0242x over previous
import functools

import jax
import jax.numpy as jnp
from jax import lax
from jax.experimental import pallas as pl
from jax.experimental.pallas import tpu as pltpu
from jax.experimental.pallas import tpu_sc as plsc

EMB = 512
BATCH = 32
SEQ = 1024
NC = 2
NS = 16
NW = NC * NS
B = BATCH * SEQ
B_PER_W = B // NW
CHUNK = 64
N_CHUNK = B_PER_W // CHUNK
NBUF = 3

_mesh = plsc.VectorSubcoreMesh(core_axis_name="c", subcore_axis_name="s")


@functools.partial(
    pl.kernel,
    mesh=_mesh,
    out_type=jax.ShapeDtypeStruct((B, EMB), jnp.float32),
    scratch_types=[
        pltpu.VMEM((N_CHUNK, CHUNK), jnp.int32),
        pltpu.VMEM((NBUF, CHUNK, EMB), jnp.float32),
        pltpu.SemaphoreType.DMA((NBUF,)),
        pltpu.SemaphoreType.DMA((NBUF,)),
    ],
)
def _emb_lookup(idx_hbm, table_hbm, out_hbm, idx_v, rows_v, gsem, wsem):
    wid = lax.axis_index("s") * NC + lax.axis_index("c")
    base = wid * B_PER_W
    pltpu.sync_copy(idx_hbm.at[wid], idx_v)

    def gather(i):
        return pltpu.async_copy(
            table_hbm.at[idx_v.at[i]], rows_v.at[i % NBUF], gsem.at[i % NBUF])

    def writeback(i):
        return pltpu.async_copy(
            rows_v.at[i % NBUF],
            out_hbm.at[pl.ds(base + i * CHUNK, CHUNK)], wsem.at[i % NBUF])

    gathers = [None] * N_CHUNK
    writes = [None] * N_CHUNK
    gathers[0] = gather(0)
    gathers[1] = gather(1)
    for i in range(N_CHUNK):
        gathers[i].wait()
        writes[i] = writeback(i)
        if i + 2 < N_CHUNK:
            if i >= 1:
                writes[i - 1].wait()
            gathers[i + 2] = gather(i + 2)
    writes[N_CHUNK - 2].wait()
    writes[N_CHUNK - 1].wait()


def kernel(x, table):
    idx = x.astype(jnp.int32).reshape(NW, N_CHUNK, CHUNK)
    out = _emb_lookup(idx, table)
    return out.reshape(BATCH, SEQ, EMB)


# speedup vs baseline: 1.6751x; 1.0501x over previous
import functools

import jax
import jax.numpy as jnp
from jax import lax
from jax.experimental import pallas as pl
from jax.experimental.pallas import tpu as pltpu
from jax.experimental.pallas import tpu_sc as plsc

EMB = 512
BATCH = 32
SEQ = 1024
NC = 2
NS = 16
NW = NC * NS
B = BATCH * SEQ
B_PER_W = B // NW
CHUNK = 64
N_CHUNK = B_PER_W // CHUNK
NBUF = 2

_mesh = plsc.VectorSubcoreMesh(core_axis_name="c", subcore_axis_name="s")


@functools.partial(
    pl.kernel,
    mesh=_mesh,
    out_type=jax.ShapeDtypeStruct((B, EMB), jnp.float32),
    scratch_types=[
        pltpu.VMEM((B_PER_W,), jnp.int32),
        pltpu.VMEM((NBUF, CHUNK, EMB), jnp.float32),
        pltpu.SemaphoreType.DMA((NBUF,)),
    ],
)
def _emb_lookup(idx_hbm, table_hbm, out_hbm, idx_v, rows_v, gsem):
    wid = lax.axis_index("s") * NC + lax.axis_index("c")
    base = wid * B_PER_W
    pltpu.sync_copy(idx_hbm.at[pl.ds(base, B_PER_W)], idx_v)

    def gather(i, b):
        return pltpu.async_copy(
            table_hbm.at[idx_v.at[pl.ds(i * CHUNK, CHUNK)]],
            rows_v.at[b], gsem.at[b])

    gather(0, 0)
    gather(1, 1)

    def body(g2, carry):
        i0 = g2 * NBUF
        for b in range(NBUF):
            i = i0 + b
            pltpu.make_async_copy(
                table_hbm.at[idx_v.at[pl.ds(i * CHUNK, CHUNK)]],
                rows_v.at[b], gsem.at[b]).wait()

            pltpu.sync_copy(rows_v.at[b],
                            out_hbm.at[pl.ds(base + i * CHUNK, CHUNK)])

            @pl.when(i + NBUF < N_CHUNK)
            def _():
                gather(i + NBUF, b)
        return carry

    lax.fori_loop(0, N_CHUNK // NBUF, body, 0)


def kernel(x, table):
    idx = x.reshape(B)
    out = _emb_lookup(idx, table)
    return out.reshape(BATCH, SEQ, EMB)
